# Initial kernel scaffold; baseline (speedup 1.0000x reference)
import jax
import jax.numpy as jnp
from jax.experimental import pallas as pl


def kernel(labels, drop_labels, embedding_table):
    raise NotImplementedError("write your pallas kernel here")


# speedup vs baseline: 1.5104x; 1.5104x over previous
import functools

import jax
import jax.numpy as jnp
from jax import lax
from jax.experimental import pallas as pl
from jax.experimental.pallas import tpu as pltpu
from jax.experimental.pallas import tpu_sc as plsc

_NULL_INDEX = 100000

_NUM_CORES = 2
_NUM_SUBCORES = 16
_LANES = 16
_NW = _NUM_CORES * _NUM_SUBCORES

_IDX_CHUNK = 128


def _build_embed(B, D):
  assert B % (8 * _NW) == 0
  bpw = B // _NW
  nchunk = bpw // _IDX_CHUNK
  assert nchunk * _IDX_CHUNK == bpw

  mesh = plsc.VectorSubcoreMesh(core_axis_name="c", subcore_axis_name="s")

  @functools.partial(
      pl.kernel,
      mesh=mesh,
      out_type=jax.ShapeDtypeStruct((B, D), jnp.float32),
      compiler_params=pltpu.CompilerParams(use_tc_tiling_on_sc=False),
      scratch_types=[
          pltpu.VMEM((bpw,), jnp.int32),
          pltpu.VMEM((bpw,), jnp.int32),
          pltpu.VMEM((nchunk, _IDX_CHUNK), jnp.int32),
          pltpu.VMEM((bpw, D), jnp.float32),
          pltpu.SemaphoreType.DMA,
      ],
  )
  def embed(labels_hbm, drop_hbm, table_hbm, out_hbm,
            lab_v, drop_v, idx_v, rows_v, sem):
    wid = lax.axis_index("s") * _NUM_CORES + lax.axis_index("c")
    base = wid * bpw
    pltpu.sync_copy(labels_hbm.at[pl.ds(base, bpw)], lab_v)
    pltpu.sync_copy(drop_hbm.at[pl.ds(base, bpw)], drop_v)
    null_vec = jnp.full((_LANES,), _NULL_INDEX, jnp.int32)
    for i in range(bpw // _LANES):
      lab = lab_v[pl.ds(i * _LANES, _LANES)]
      drp = drop_v[pl.ds(i * _LANES, _LANES)]
      masked = jnp.where(drp != 0, null_vec, lab)
      row = i // (_IDX_CHUNK // _LANES)
      col = (i % (_IDX_CHUNK // _LANES)) * _LANES
      idx_v[row, pl.ds(col, _LANES)] = masked
    copies = []
    for j in range(nchunk):
      copies.append(
          pltpu.async_copy(
              table_hbm.at[idx_v.at[j]],
              rows_v.at[pl.ds(j * _IDX_CHUNK, _IDX_CHUNK)],
              sem,
          ))
    for c in copies:
      c.wait()
    pltpu.sync_copy(rows_v, out_hbm.at[pl.ds(base, bpw)])

  return embed


@jax.jit
def _embed_call(labels, drop, table):
  B, = labels.shape
  _, D = table.shape
  return _build_embed(B, D)(labels, drop, table)


def kernel(labels, drop_labels, embedding_table):
  labels = labels.astype(jnp.int32)
  drop = drop_labels.astype(jnp.int32)
  return _embed_call(labels, drop, embedding_table)


# speedup vs baseline: 3.7216x; 2.4640x over previous
import functools

import jax
import jax.numpy as jnp
from jax import lax
from jax.experimental import pallas as pl
from jax.experimental.pallas import tpu as pltpu
from jax.experimental.pallas import tpu_sc as plsc

_NULL_INDEX = 100000

_NUM_CORES = 2
_NUM_SUBCORES = 16
_LANES = 16
_NW = _NUM_CORES * _NUM_SUBCORES

_IDX_CHUNK = 128


def _build_embed(B, D):
  assert B % (8 * _NW) == 0
  bpw = B // _NW
  nchunk = bpw // _IDX_CHUNK
  assert nchunk * _IDX_CHUNK == bpw
  ncg = D // _LANES

  mesh = plsc.VectorSubcoreMesh(core_axis_name="c", subcore_axis_name="s")

  @functools.partial(
      pl.kernel,
      mesh=mesh,
      out_type=jax.ShapeDtypeStruct((B, D), jnp.float32),
      compiler_params=pltpu.CompilerParams(use_tc_tiling_on_sc=False,
                                           needs_layout_passes=False),
      scratch_types=[
          pltpu.VMEM((bpw,), jnp.int32),
          pltpu.VMEM((nchunk, _IDX_CHUNK), jnp.int32),
          pltpu.VMEM((1, D), jnp.float32),
          pltpu.VMEM((bpw, D), jnp.float32),
          pltpu.SemaphoreType.DMA,
      ],
  )
  def embed(labels_hbm, drop_hbm, table_hbm, out_hbm,
            drop_v, idx_v, null_v, rows_v, sem):
    wid = lax.axis_index("s") * _NUM_CORES + lax.axis_index("c")
    base = wid * bpw
    for j in range(nchunk):
      pltpu.sync_copy(labels_hbm.at[pl.ds(base + j * _IDX_CHUNK, _IDX_CHUNK)],
                      idx_v.at[j])
    pltpu.sync_copy(drop_hbm.at[pl.ds(base, bpw)], drop_v)
    pltpu.sync_copy(table_hbm.at[pl.ds(_NULL_INDEX, 1)], null_v)
    copies = []
    for j in range(nchunk):
      copies.append(
          pltpu.async_copy(
              table_hbm.at[idx_v.at[j]],
              rows_v.at[pl.ds(j * _IDX_CHUNK, _IDX_CHUNK)],
              sem,
          ))
    for c in copies:
      c.wait()

    null_cg = [null_v[0, pl.ds(cg * _LANES, _LANES)] for cg in range(ncg)]

    def blend_group(i, carry):
      for j in range(_LANES):
        r = i * _LANES + j
        flag = plsc.load_gather(drop_v, [jnp.full((_LANES,), r, jnp.int32)])
        pred = flag != 0
        for cg in range(ncg):
          cur = rows_v[r, pl.ds(cg * _LANES, _LANES)]
          rows_v[r, pl.ds(cg * _LANES, _LANES)] = jnp.where(
              pred, null_cg[cg], cur)
      return carry

    lax.fori_loop(0, bpw // _LANES, blend_group, 0)
    pltpu.sync_copy(rows_v, out_hbm.at[pl.ds(base, bpw)])

  return embed


@jax.jit
def _embed_call(labels, drop, table):
  B, = labels.shape
  _, D = table.shape
  return _build_embed(B, D)(labels, drop, table)


def kernel(labels, drop_labels, embedding_table):
  labels = labels.astype(jnp.int32)
  drop = drop_labels.astype(jnp.int32)
  return _embed_call(labels, drop, embedding_table)
